# Initial kernel scaffold; baseline (speedup 1.0000x reference)
import jax
import jax.numpy as jnp
from jax.experimental import pallas as pl


def kernel(indices, C0, C1):
    raise NotImplementedError("write your pallas kernel here")


# speedup vs baseline: 3.6600x; 3.6600x over previous
import functools

import jax
import jax.numpy as jnp
from jax import lax
from jax.experimental import pallas as pl
from jax.experimental.pallas import tpu as pltpu
from jax.experimental.pallas import tpu_sc as plsc

B = 16384
N = 1000
R = 64
NC = 2
NS = 16
NW = NC * NS
BPW = B // NW
CH = 128
NCH = BPW // CH
L = 16


def _mm_body(t0_ref, t1_ref, g_ref):
    g_ref[...] = jnp.dot(
        t0_ref[...], t1_ref[...],
        preferred_element_type=jnp.float32,
        precision=lax.Precision.HIGHEST,
    )


def _make_gather_kernel():
    mesh = plsc.VectorSubcoreMesh(core_axis_name="c", subcore_axis_name="s")

    @functools.partial(
        pl.kernel,
        mesh=mesh,
        out_type=jax.ShapeDtypeStruct((B,), jnp.float32),
        scratch_types=[
            pltpu.VMEM((NCH, CH), jnp.int32),
            pltpu.VMEM((NCH, CH), jnp.int32),
            pltpu.VMEM((NCH, CH), jnp.int32),
            pltpu.VMEM((BPW,), jnp.float32),
            pltpu.SemaphoreType.DMA,
        ],
    )
    def k(idx0_hbm, idx1_hbm, g_hbm, out_hbm, i0_v, i1_v, f_v, o_v, sem):
        wid = lax.axis_index("s") * NC + lax.axis_index("c")
        cbase = wid * NCH

        pltpu.sync_copy(idx0_hbm.at[pl.ds(cbase, NCH)], i0_v)
        pltpu.sync_copy(idx1_hbm.at[pl.ds(cbase, NCH)], i1_v)

        for c in range(NCH):
            for l in range(CH // L):
                a = i0_v[c, pl.ds(l * L, L)]
                b = i1_v[c, pl.ds(l * L, L)]
                f_v[c, pl.ds(l * L, L)] = a * N + b

        copies = [
            pltpu.async_copy(g_hbm.at[f_v.at[c]],
                             o_v.at[pl.ds(c * CH, CH)], sem)
            for c in range(NCH)
        ]
        for cp in copies:
            cp.wait()

        pltpu.sync_copy(o_v, out_hbm.at[pl.ds(wid * BPW, BPW)])

    return k


_gather_kernel = _make_gather_kernel()


def kernel(indices, C0, C1):
    idx = indices.astype(jnp.int32)
    idx0 = idx[:, 0].reshape(NW * NCH, CH)
    idx1 = idx[:, 1].reshape(NW * NCH, CH)
    t0 = C0[0]
    t1 = C1[:, :, 0]

    g = pl.pallas_call(
        _mm_body,
        out_shape=jax.ShapeDtypeStruct((N, N), jnp.float32),
    )(t0, t1)

    return _gather_kernel(idx0, idx1, g.reshape(N * N), )


# speedup vs baseline: 3.9334x; 1.0747x over previous
import functools

import jax
import jax.numpy as jnp
from jax import lax
from jax.experimental import pallas as pl
from jax.experimental.pallas import tpu as pltpu
from jax.experimental.pallas import tpu_sc as plsc

B = 16384
N = 1000
NP = 1024
R = 64
NC = 2
NS = 16
NW = NC * NS
BPW = B // NW
CH = 128
NCH = BPW // CH
L = 16


def _mm_body(t0_ref, t1_ref, g_ref):
    g_ref[...] = jnp.dot(
        t0_ref[...], t1_ref[...],
        preferred_element_type=jnp.float32,
        precision=lax.Precision.HIGHEST,
    )[None]


def _make_gather_kernel():
    mesh = plsc.VectorSubcoreMesh(core_axis_name="c", subcore_axis_name="s")

    @functools.partial(
        pl.kernel,
        mesh=mesh,
        out_type=jax.ShapeDtypeStruct((B,), jnp.float32),
        scratch_types=[
            pltpu.VMEM((BPW,), jnp.int32),
            pltpu.VMEM((BPW,), jnp.int32),
            pltpu.VMEM((BPW,), jnp.int32),
            pltpu.VMEM((BPW,), jnp.float32),
            pltpu.SemaphoreType.DMA,
        ],
    )
    def k(idx0_hbm, idx1_hbm, g_hbm, out_hbm, i0_v, i1_v, f_v, o_v, sem):
        wid = lax.axis_index("s") * NC + lax.axis_index("c")

        pltpu.sync_copy(idx0_hbm.at[wid], i0_v)
        pltpu.sync_copy(idx1_hbm.at[wid], i1_v)

        def vec_body(vi, _):
            s = pl.ds(vi * L, L)
            i = i0_v[s]
            j = i1_v[s]
            f_v[s] = (lax.shift_right_logical(j, 7) * (N * 128)
                      + i * 128 + lax.bitwise_and(j, 127))
            return 0

        lax.fori_loop(0, BPW // L, vec_body, 0)

        copies = [
            pltpu.async_copy(g_hbm.at[f_v.at[pl.ds(c * CH, CH)]],
                             o_v.at[pl.ds(c * CH, CH)], sem)
            for c in range(NCH)
        ]
        for cp in copies:
            cp.wait()

        pltpu.sync_copy(o_v, out_hbm.at[pl.ds(wid * BPW, BPW)])

    return k


_gather_kernel = _make_gather_kernel()


def kernel(indices, C0, C1):
    idx = indices.astype(jnp.int32)
    idx0 = idx[:, 0].reshape(NW, BPW)
    idx1 = idx[:, 1].reshape(NW, BPW)
    t0 = C0[0]
    t1 = jnp.pad(C1[:, :, 0], ((0, 0), (0, NP - N)))

    g = pl.pallas_call(
        _mm_body,
        grid=(NP // 128,),
        in_specs=[
            pl.BlockSpec((N, R), lambda ct: (0, 0)),
            pl.BlockSpec((R, 128), lambda ct: (0, ct)),
        ],
        out_specs=pl.BlockSpec((1, N, 128), lambda ct: (ct, 0, 0)),
        out_shape=jax.ShapeDtypeStruct((NP // 128, N, 128), jnp.float32),
    )(t0, t1)

    return _gather_kernel(idx0, idx1, g.reshape(NP * N))


# speedup vs baseline: 4.3410x; 1.1036x over previous
import functools

import jax
import jax.numpy as jnp
from jax import lax
from jax.experimental import pallas as pl
from jax.experimental.pallas import tpu as pltpu
from jax.experimental.pallas import tpu_sc as plsc

B = 16384
N = 1000
NP = 1024
R = 64
NC = 2
NS = 16
NW = NC * NS
BPW = B // NW
CH = 128
NCH = BPW // CH
L = 16


def _mm_body(t0_ref, t1_ref, g_ref):
    g_ref[...] = jnp.dot(
        t0_ref[...], t1_ref[...],
        preferred_element_type=jnp.float32,
        precision=lax.Precision.DEFAULT,
    )[None]


def _make_gather_kernel():
    mesh = plsc.VectorSubcoreMesh(core_axis_name="c", subcore_axis_name="s")

    @functools.partial(
        pl.kernel,
        mesh=mesh,
        out_type=jax.ShapeDtypeStruct((B,), jnp.float32),
        scratch_types=[
            pltpu.VMEM((BPW,), jnp.int32),
            pltpu.VMEM((BPW,), jnp.int32),
            pltpu.VMEM((BPW,), jnp.int32),
            pltpu.VMEM((BPW,), jnp.float32),
            pltpu.SemaphoreType.DMA,
        ],
    )
    def k(idx0_hbm, idx1_hbm, g_hbm, out_hbm, i0_v, i1_v, f_v, o_v, sem):
        wid = lax.axis_index("s") * NC + lax.axis_index("c")

        pltpu.sync_copy(idx0_hbm.at[wid], i0_v)
        pltpu.sync_copy(idx1_hbm.at[wid], i1_v)

        def vec_body(vi, _):
            s = pl.ds(vi * L, L)
            i = i0_v[s]
            j = i1_v[s]
            f_v[s] = (lax.shift_right_logical(j, 7) * (N * 128)
                      + i * 128 + lax.bitwise_and(j, 127))
            return 0

        lax.fori_loop(0, BPW // L, vec_body, 0)

        copies = [
            pltpu.async_copy(g_hbm.at[f_v.at[pl.ds(c * CH, CH)]],
                             o_v.at[pl.ds(c * CH, CH)], sem)
            for c in range(NCH)
        ]
        for cp in copies:
            cp.wait()

        pltpu.sync_copy(o_v, out_hbm.at[pl.ds(wid * BPW, BPW)])

    return k


_gather_kernel = _make_gather_kernel()


def kernel(indices, C0, C1):
    idx = indices.astype(jnp.int32)
    idx0 = idx[:, 0].reshape(NW, BPW)
    idx1 = idx[:, 1].reshape(NW, BPW)
    t0 = C0[0]
    t1 = jnp.pad(C1[:, :, 0], ((0, 0), (0, NP - N)))

    g = pl.pallas_call(
        _mm_body,
        grid=(NP // 128,),
        in_specs=[
            pl.BlockSpec((N, R), lambda ct: (0, 0)),
            pl.BlockSpec((R, 128), lambda ct: (0, ct)),
        ],
        out_specs=pl.BlockSpec((1, N, 128), lambda ct: (ct, 0, 0)),
        out_shape=jax.ShapeDtypeStruct((NP // 128, N, 128), jnp.float32),
    )(t0, t1)

    return _gather_kernel(idx0, idx1, g.reshape(NP * N))


# speedup vs baseline: 5.1044x; 1.1759x over previous
import functools

import jax
import jax.numpy as jnp
from jax import lax
from jax.experimental import pallas as pl
from jax.experimental.pallas import tpu as pltpu
from jax.experimental.pallas import tpu_sc as plsc

B = 16384
N = 1000
NP = 1024
R = 64
NC = 2
NS = 16
NW = NC * NS
BPW = B // NW
CH = 128
NCH = BPW // CH
L = 16


def _mm_body(t0_ref, t1_ref, g_ref):
    t0 = t0_ref[...]
    for t in range(NP // 128):
        g_ref[t] = jnp.dot(
            t0, t1_ref[:, t * 128:(t + 1) * 128],
            preferred_element_type=jnp.float32,
            precision=lax.Precision.DEFAULT,
        )


def _make_gather_kernel():
    mesh = plsc.VectorSubcoreMesh(core_axis_name="c", subcore_axis_name="s")

    @functools.partial(
        pl.kernel,
        mesh=mesh,
        out_type=jax.ShapeDtypeStruct((B,), jnp.float32),
        scratch_types=[
            pltpu.VMEM((BPW,), jnp.int32),
            pltpu.VMEM((BPW,), jnp.int32),
            pltpu.VMEM((BPW,), jnp.int32),
            pltpu.VMEM((BPW,), jnp.float32),
            pltpu.SemaphoreType.DMA,
        ],
    )
    def k(idxt_hbm, g_hbm, out_hbm, i0_v, i1_v, f_v, o_v, sem):
        wid = lax.axis_index("s") * NC + lax.axis_index("c")

        pltpu.sync_copy(idxt_hbm.at[0, pl.ds(wid * BPW, BPW)], i0_v)
        pltpu.sync_copy(idxt_hbm.at[1, pl.ds(wid * BPW, BPW)], i1_v)

        def vec_body(vi, _):
            s = pl.ds(vi * L, L)
            i = i0_v[s]
            j = i1_v[s]
            f_v[s] = (lax.shift_right_logical(j, 7) * (N * 128)
                      + i * 128 + lax.bitwise_and(j, 127))
            return 0

        lax.fori_loop(0, BPW // L, vec_body, 0)

        copies = [
            pltpu.async_copy(g_hbm.at[f_v.at[pl.ds(c * CH, CH)]],
                             o_v.at[pl.ds(c * CH, CH)], sem)
            for c in range(NCH)
        ]
        for cp in copies:
            cp.wait()

        pltpu.sync_copy(o_v, out_hbm.at[pl.ds(wid * BPW, BPW)])

    return k


_gather_kernel = _make_gather_kernel()


def kernel(indices, C0, C1):
    idxt = jnp.transpose(indices.astype(jnp.int32))
    t0 = C0[0]
    t1 = jnp.pad(C1[:, :, 0], ((0, 0), (0, NP - N)))

    g = pl.pallas_call(
        _mm_body,
        out_shape=jax.ShapeDtypeStruct((NP // 128, N, 128), jnp.float32),
    )(t0, t1)

    return _gather_kernel(idxt, g.reshape(NP * N))


# speedup vs baseline: 5.6382x; 1.1046x over previous
import functools

import jax
import jax.numpy as jnp
from jax import lax
from jax.experimental import pallas as pl
from jax.experimental.pallas import tpu as pltpu
from jax.experimental.pallas import tpu_sc as plsc

B = 16384
N = 1000
NP = 1024
R = 64
NC = 2
NS = 16
NW = NC * NS
BPW = B // NW
CH = 128
NCH = BPW // CH
L = 16


def _mm_body(t0t_ref, t1_ref, g_ref):
    t0t = t0t_ref[...]
    for t in range(NP // 128):
        lo = t * 128
        hi = min(lo + 128, N)
        d = lax.dot_general(
            t0t, t1_ref[:, lo:hi],
            dimension_numbers=(((0,), (0,)), ((), ())),
            preferred_element_type=jnp.float32,
            precision=lax.Precision.DEFAULT,
        )
        if hi - lo < 128:
            d = jnp.pad(d, ((0, 0), (0, 128 - (hi - lo))))
        g_ref[t] = d


def _make_gather_kernel():
    mesh = plsc.VectorSubcoreMesh(core_axis_name="c", subcore_axis_name="s")

    @functools.partial(
        pl.kernel,
        mesh=mesh,
        out_type=jax.ShapeDtypeStruct((B,), jnp.float32),
        scratch_types=[
            pltpu.VMEM((2, BPW), jnp.int32),
            pltpu.VMEM((BPW,), jnp.int32),
            pltpu.VMEM((BPW,), jnp.float32),
            pltpu.SemaphoreType.DMA,
        ],
    )
    def k(idxt_hbm, g_hbm, out_hbm, i01_v, f_v, o_v, sem):
        wid = lax.axis_index("s") * NC + lax.axis_index("c")

        pltpu.sync_copy(idxt_hbm.at[:, pl.ds(wid * BPW, BPW)], i01_v)

        copies = []
        for c in range(NCH):
            def vec_body(vi, _, c=c):
                s = pl.ds(c * CH + vi * L, L)
                i = i01_v[0, s]
                j = i01_v[1, s]
                f_v[s] = (lax.shift_right_logical(j, 7) * (N * 128)
                          + i * 128 + lax.bitwise_and(j, 127))
                return 0

            lax.fori_loop(0, CH // L, vec_body, 0)
            copies.append(
                pltpu.async_copy(g_hbm.at[f_v.at[pl.ds(c * CH, CH)]],
                                 o_v.at[pl.ds(c * CH, CH)], sem))
        for cp in copies:
            cp.wait()

        pltpu.sync_copy(o_v, out_hbm.at[pl.ds(wid * BPW, BPW)])

    return k


_gather_kernel = _make_gather_kernel()


def kernel(indices, C0, C1):
    idxt = jnp.transpose(indices.astype(jnp.int32))
    t0t = jnp.transpose(C0[0])
    t1 = C1[:, :, 0]

    g = pl.pallas_call(
        _mm_body,
        out_shape=jax.ShapeDtypeStruct((NP // 128, N, 128), jnp.float32),
    )(t0t, t1)

    return _gather_kernel(idxt, g.reshape(NP * N))


# speedup vs baseline: 5.6510x; 1.0023x over previous
import functools

import jax
import jax.numpy as jnp
from jax import lax
from jax.experimental import pallas as pl
from jax.experimental.pallas import tpu as pltpu
from jax.experimental.pallas import tpu_sc as plsc

B = 16384
N = 1000
NP = 1024
R = 64
NC = 2
NS = 16
NW = NC * NS
BPW = B // NW
CH = 128
NCH = BPW // CH
L = 16


def _mm_body(t0t_ref, t1_ref, g_ref):
    t0t = t0t_ref[...]
    for t in range(NP // 128):
        lo = t * 128
        hi = min(lo + 128, N)
        d = lax.dot_general(
            t0t, t1_ref[:, lo:hi],
            dimension_numbers=(((0,), (0,)), ((), ())),
            preferred_element_type=jnp.float32,
            precision=lax.Precision.DEFAULT,
        )
        if hi - lo < 128:
            d = jnp.pad(d, ((0, 0), (0, 128 - (hi - lo))))
        g_ref[t] = d


def _make_gather_kernel():
    mesh = plsc.VectorSubcoreMesh(core_axis_name="c", subcore_axis_name="s")

    @functools.partial(
        pl.kernel,
        mesh=mesh,
        out_type=jax.ShapeDtypeStruct((B,), jnp.float32),
        scratch_types=[
            pltpu.VMEM((2, BPW), jnp.int32),
            pltpu.VMEM((BPW,), jnp.int32),
            pltpu.VMEM((BPW,), jnp.float32),
            pltpu.SemaphoreType.DMA,
        ],
    )
    def k(idxt_hbm, g_hbm, out_hbm, i01_v, f_v, o_v, sem):
        wid = lax.axis_index("s") * NC + lax.axis_index("c")

        pltpu.sync_copy(idxt_hbm.at[:, pl.ds(wid * BPW, BPW)], i01_v)

        copies = []
        for c in range(NCH):
            for vi in range(CH // L):
                s = pl.ds(c * CH + vi * L, L)
                i = i01_v[0, s]
                j = i01_v[1, s]
                f_v[s] = (lax.shift_right_logical(j, 7) * (N * 128)
                          + i * 128 + lax.bitwise_and(j, 127))
            copies.append(
                pltpu.async_copy(g_hbm.at[f_v.at[pl.ds(c * CH, CH)]],
                                 o_v.at[pl.ds(c * CH, CH)], sem))
        for cp in copies:
            cp.wait()

        pltpu.sync_copy(o_v, out_hbm.at[pl.ds(wid * BPW, BPW)])

    return k


_gather_kernel = _make_gather_kernel()


def kernel(indices, C0, C1):
    idxt = jnp.transpose(indices.astype(jnp.int32))
    t0t = jnp.transpose(C0[0])

    g = pl.pallas_call(
        _mm_body,
        out_shape=jax.ShapeDtypeStruct((NP // 128, N, 128), jnp.float32),
    )(t0t, C1[:, :, 0])

    return _gather_kernel(idxt, g.reshape(NP * N))
